# baseline (speedup 1.0000x reference)
import jax
import jax.numpy as jnp
from jax.experimental import pallas as pl
from jax.experimental.pallas import tpu as pltpu

_KC = 512


def _vq_argmin_kernel(x_hbm, c_hbm, out_ref, xbuf, cbuf, sem_x, sem_c):
    k = c_hbm.shape[0]
    nchunks = k // _KC

    x_copy = pltpu.make_async_copy(x_hbm, xbuf, sem_x)
    x_copy.start()

    def c_copy(j):
        return pltpu.make_async_copy(
            c_hbm.at[pl.ds(j * _KC, _KC), :], cbuf.at[j], sem_c.at[j])

    for j in range(nchunks):
        c_copy(j).start()

    m = None
    idx = None
    for j in range(nchunks):
        c_copy(j).wait()
        ct2_j = cbuf[j].T * -2.0
        cn_j = 0.25 * jnp.sum(ct2_j * ct2_j, axis=0, keepdims=True)
        if j == 0:
            x_copy.wait()
        g_j = jnp.dot(xbuf[...], ct2_j, preferred_element_type=jnp.float32,
                      precision=jax.lax.Precision.HIGHEST)
        score_j = cn_j + g_j
        m_j = jnp.min(score_j, axis=1, keepdims=True)
        col = jax.lax.broadcasted_iota(jnp.int32, score_j.shape, 1) + j * _KC
        idx_j = jnp.min(jnp.where(score_j == m_j, col, k),
                        axis=1, keepdims=True)
        if m is None:
            m, idx = m_j, idx_j
        else:
            better = m_j < m
            idx = jnp.where(better, idx_j, idx)
            m = jnp.minimum(m_j, m)
    out_ref[...] = idx[:, 0]


def kernel(x, centroids):
    n, d = x.shape
    k = centroids.shape[0]
    return pl.pallas_call(
        _vq_argmin_kernel,
        in_specs=[
            pl.BlockSpec(memory_space=pltpu.MemorySpace.HBM),
            pl.BlockSpec(memory_space=pltpu.MemorySpace.HBM),
        ],
        out_shape=jax.ShapeDtypeStruct((n,), jnp.int32),
        scratch_shapes=[
            pltpu.VMEM((n, d), jnp.float32),
            pltpu.VMEM((k // _KC, _KC, d), jnp.float32),
            pltpu.SemaphoreType.DMA,
            pltpu.SemaphoreType.DMA((2,)),
        ],
    )(x, centroids)


# speedup vs baseline: 1.0961x; 1.0961x over previous
import jax
import jax.numpy as jnp
from jax.experimental import pallas as pl

_LB = 128


def _vq_argmin_kernel(x_ref, c_ref, out_ref):
    ct2 = c_ref[...].T * -2.0
    k = ct2.shape[1]
    cnorm = 0.25 * jnp.sum(ct2 * ct2, axis=0, keepdims=True)
    g2 = jnp.dot(x_ref[...], ct2, preferred_element_type=jnp.float32,
                 precision=jax.lax.Precision.HIGHEST)

    m8 = cnorm[:, :_LB] + g2[:, :_LB]
    a8 = jnp.zeros(m8.shape, jnp.int32)
    for c in range(1, k // _LB):
        s_c = cnorm[:, c * _LB:(c + 1) * _LB] + g2[:, c * _LB:(c + 1) * _LB]
        lt = s_c < m8
        m8 = jnp.where(lt, s_c, m8)
        a8 = jnp.where(lt, c, a8)
    m = jnp.min(m8, axis=1, keepdims=True)
    lane = jax.lax.broadcasted_iota(jnp.int32, m8.shape, 1)
    idxp = a8 * _LB + lane
    idx = jnp.min(jnp.where(m8 == m, idxp, k), axis=1)
    out_ref[...] = idx


def kernel(x, centroids):
    n, d = x.shape
    return pl.pallas_call(
        _vq_argmin_kernel,
        out_shape=jax.ShapeDtypeStruct((n,), jnp.int32),
    )(x, centroids)


# speedup vs baseline: 1.3393x; 1.2219x over previous
import jax
import jax.numpy as jnp
from jax.experimental import pallas as pl

_LB = 128


def _vq_argmin_kernel(x_ref, c_ref, out_ref):
    ct2 = c_ref[...].T * -2.0
    k = ct2.shape[1]
    cnorm = 0.25 * jnp.sum(ct2 * ct2, axis=0, keepdims=True)
    g2 = jnp.dot(x_ref[...], ct2, preferred_element_type=jnp.float32,
                 precision=jax.lax.Precision.HIGHEST)

    m8 = cnorm[:, :_LB] + g2[:, :_LB]
    a8 = jnp.zeros(m8.shape, jnp.int32)
    for c in range(1, k // _LB):
        s_c = cnorm[:, c * _LB:(c + 1) * _LB] + g2[:, c * _LB:(c + 1) * _LB]
        lt = s_c < m8
        m8 = jnp.where(lt, s_c, m8)
        a8 = jnp.where(lt, c, a8)
    lane = jax.lax.broadcasted_iota(jnp.int32, m8.shape, 1)
    idxp = a8 * _LB + lane
    m_t = m8.T
    i_t = idxp.T
    m = jnp.min(m_t, axis=0, keepdims=True)
    idx = jnp.min(jnp.where(m_t == m, i_t, k), axis=0)
    out_ref[...] = idx


def kernel(x, centroids):
    n, d = x.shape
    return pl.pallas_call(
        _vq_argmin_kernel,
        out_shape=jax.ShapeDtypeStruct((n,), jnp.int32),
    )(x, centroids)
